# Initial kernel scaffold; baseline (speedup 1.0000x reference)
import jax
import jax.numpy as jnp
from jax.experimental import pallas as pl


def kernel(atom_number, distance, atom_membership, distance_membership_i, distance_membership_j, embedding, W_cf_0, b_cf_0, W_df_0, b_df_0, W_fc_0, W_cf_1, b_cf_1, W_df_1, b_df_1, W_fc_1, W_g1, b_g1, W_gout, b_gout, W_lin, b_lin):
    raise NotImplementedError("write your pallas kernel here")


# speedup vs baseline: 3.6882x; 3.6882x over previous
import functools

import jax
import jax.numpy as jnp
from jax import lax
from jax.experimental import pallas as pl
from jax.experimental.pallas import tpu as pltpu
from jax.experimental.pallas import tpu_sc as plsc

N_MOLS = 2500

BLK_E = 2000
BLK_N = 2000
WIN_A = 128
WIN_M = 128


def _pad2(x, r, c):
  return jnp.pad(x, ((0, r - x.shape[0]), (0, c - x.shape[1])))


def _pad1(x, n):
  return jnp.pad(x, (0, n - x.shape[0]))


def _sc_gather_i32(table, idx):
  E = idx.shape[0]
  mesh = plsc.VectorSubcoreMesh(core_axis_name="c", subcore_axis_name="s")
  nw = mesh.num_cores * mesh.num_subcores
  per = E // nw
  C = 5000
  assert per % C == 0 and per % 8 == 0

  @functools.partial(
      pl.kernel,
      out_type=jax.ShapeDtypeStruct((E,), jnp.int32),
      mesh=mesh,
      scratch_types=[
          pltpu.VMEM((C,), jnp.int32),
          pltpu.VMEM((C,), jnp.int32),
          pltpu.SemaphoreType.DMA,
      ],
  )
  def k(table_hbm, idx_hbm, out_hbm, idx_v, rows_v, sem):
    wid = lax.axis_index("s") * mesh.num_cores + lax.axis_index("c")
    base = wid * per

    def body(i, carry):
      off = base + i * C
      pltpu.sync_copy(idx_hbm.at[pl.ds(off, C)], idx_v)
      pltpu.async_copy(table_hbm.at[idx_v], rows_v, sem).wait()
      pltpu.sync_copy(rows_v, out_hbm.at[pl.ds(off, C)])
      return carry

    lax.fori_loop(0, per // C, body, 0)

  return k(table, idx)


def _sc_gather_rows(table, idx, d):
  E = idx.shape[0]
  mesh = plsc.VectorSubcoreMesh(core_axis_name="c", subcore_axis_name="s")
  nw = mesh.num_cores * mesh.num_subcores
  per = E // nw
  C = 200
  assert per % C == 0 and C % 8 == 0 and per % 8 == 0

  @functools.partial(
      pl.kernel,
      out_type=jax.ShapeDtypeStruct((E, d), jnp.float32),
      mesh=mesh,
      scratch_types=[
          pltpu.VMEM((C,), jnp.int32),
          pltpu.VMEM((C, d), jnp.float32),
          pltpu.SemaphoreType.DMA,
      ],
  )
  def k(table_hbm, idx_hbm, out_hbm, idx_v, rows_v, sem):
    wid = lax.axis_index("s") * mesh.num_cores + lax.axis_index("c")
    base = wid * per

    def body(i, carry):
      off = base + i * C
      pltpu.sync_copy(idx_hbm.at[pl.ds(off, C)], idx_v)
      pltpu.async_copy(table_hbm.at[idx_v], rows_v, sem).wait()
      pltpu.sync_copy(rows_v, out_hbm.at[pl.ds(off, C)])
      return carry

    lax.fori_loop(0, per // C, body, 0)

  return k(table, idx)


def _window_scatter(acc_ref, ids2, vals, win):
  lo = jnp.min(ids2)
  hi = jnp.max(ids2)
  aw0 = (lo // 8) * 8
  n_win = (hi - aw0) // win + 1

  def body(w, carry):
    aw = aw0 + w * win
    rows = lax.broadcasted_iota(jnp.int32, (win, ids2.shape[1]), 0) + aw
    m = (rows == ids2).astype(jnp.float32)
    contrib = jnp.dot(m, vals, preferred_element_type=jnp.float32)
    acc_ref[pl.ds(aw, win), :] += contrib
    return carry

  lax.fori_loop(0, n_win, body, 0)


def _edge0_body(dist_ref, anj_ref, dmi_ref, emb_ref, wcf_ref, bcf_ref,
                wdf_ref, bdf_ref, wfc_ref, acc_ref):
  @pl.when(pl.program_id(0) == 0)
  def _init():
    acc_ref[...] = jnp.zeros_like(acc_ref)

  afh_t = (
      jnp.dot(emb_ref[...], wcf_ref[...], preferred_element_type=jnp.float32)
      + bcf_ref[...]
  )
  onehot = (
      lax.broadcasted_iota(jnp.int32, (BLK_E, 32), 1) == anj_ref[...]
  ).astype(jnp.float32)
  gj = jnp.dot(onehot, afh_t, preferred_element_type=jnp.float32)
  dh = (
      jnp.dot(dist_ref[...], wdf_ref[...], preferred_element_type=jnp.float32)
      + bdf_ref[...]
  )
  out = jnp.tanh(
      jnp.dot(dh * gj, wfc_ref[...], preferred_element_type=jnp.float32)
  )
  _window_scatter(acc_ref, dmi_ref[0], out, WIN_A)


def _edge1_body(dist_ref, gj_ref, dmi_ref, wdf_ref, bdf_ref, wfc_ref,
                acc_ref):
  @pl.when(pl.program_id(0) == 0)
  def _init():
    acc_ref[...] = jnp.zeros_like(acc_ref)

  dh = (
      jnp.dot(dist_ref[...], wdf_ref[...], preferred_element_type=jnp.float32)
      + bdf_ref[...]
  )
  gj = gj_ref[...][:, :64]
  out = jnp.tanh(
      jnp.dot(dh * gj, wfc_ref[...], preferred_element_type=jnp.float32)
  )
  _window_scatter(acc_ref, dmi_ref[0], out, WIN_A)


def _mid_body(agg_ref, an_ref, emb_ref, wcf0_ref, bcf0_ref, bdf0_ref,
              wfc0_ref, wcf1_ref, bcf1_ref, bdf1_ref, wfc1_ref,
              afh1_ref, base1_ref):
  afh0_t = (
      jnp.dot(emb_ref[...], wcf0_ref[...], preferred_element_type=jnp.float32)
      + bcf0_ref[...]
  )
  out_ii0_t = jnp.tanh(
      jnp.dot(
          bdf0_ref[...] * afh0_t, wfc0_ref[...],
          preferred_element_type=jnp.float32,
      )
  )
  delta_t = emb_ref[...] - out_ii0_t
  onehot = (
      lax.broadcasted_iota(jnp.int32, (BLK_N, 32), 1) == an_ref[...]
  ).astype(jnp.float32)
  af1 = agg_ref[...] + jnp.dot(
      onehot, delta_t, preferred_element_type=jnp.float32
  )
  afh1 = (
      jnp.dot(af1, wcf1_ref[...], preferred_element_type=jnp.float32)
      + bcf1_ref[...]
  )
  afh1_ref[...] = afh1
  out_ii1 = jnp.tanh(
      jnp.dot(
          bdf1_ref[...] * afh1, wfc1_ref[...],
          preferred_element_type=jnp.float32,
      )
  )
  base1_ref[...] = af1 - out_ii1


def _final_body(agg_ref, base_ref, am_ref, wg1_ref, bg1_ref, wgout_ref,
                bgout_ref, wlint_ref, blin_ref, acc_ref):
  @pl.when(pl.program_id(0) == 0)
  def _init():
    acc_ref[...] = jnp.zeros_like(acc_ref)

  af2 = agg_ref[...] + base_ref[...]
  g = jnp.tanh(
      jnp.dot(af2, wg1_ref[...], preferred_element_type=jnp.float32)
      + bg1_ref[...]
  )
  g2 = jnp.tanh(
      jnp.dot(g, wgout_ref[...], preferred_element_type=jnp.float32)
      + bgout_ref[...]
  )
  h = jnp.dot(g2, wlint_ref[...], preferred_element_type=jnp.float32)
  _window_scatter(acc_ref, am_ref[0], h, WIN_M)

  @pl.when(pl.program_id(0) == pl.num_programs(0) - 1)
  def _bias():
    acc_ref[...] += blin_ref[...]


def _full_spec(shape):
  return pl.BlockSpec(shape, lambda i: tuple(0 for _ in shape))


def _edge_pass0(distance, anj3, dmi3, emb_p, wcf_p, bcf_p, wdf_p, bdf_p,
                wfc_p, n_atoms):
  E = distance.shape[0]
  nb = E // BLK_E
  acc_rows = n_atoms + WIN_A
  return pl.pallas_call(
      _edge0_body,
      grid=(nb,),
      in_specs=[
          pl.BlockSpec((BLK_E, distance.shape[1]), lambda i: (i, 0)),
          pl.BlockSpec((BLK_E, 1), lambda i: (i, 0)),
          pl.BlockSpec((1, 1, BLK_E), lambda i: (i, 0, 0)),
          _full_spec(emb_p.shape),
          _full_spec(wcf_p.shape),
          _full_spec(bcf_p.shape),
          _full_spec(wdf_p.shape),
          _full_spec(bdf_p.shape),
          _full_spec(wfc_p.shape),
      ],
      out_specs=pl.BlockSpec((acc_rows, 32), lambda i: (0, 0)),
      out_shape=jax.ShapeDtypeStruct((acc_rows, 32), jnp.float32),
  )(distance, anj3, dmi3, emb_p, wcf_p, bcf_p, wdf_p, bdf_p, wfc_p)


def _edge_pass1(distance, gj, dmi3, wdf_p, bdf_p, wfc_p, n_atoms):
  E = distance.shape[0]
  nb = E // BLK_E
  acc_rows = n_atoms + WIN_A
  return pl.pallas_call(
      _edge1_body,
      grid=(nb,),
      in_specs=[
          pl.BlockSpec((BLK_E, distance.shape[1]), lambda i: (i, 0)),
          pl.BlockSpec((BLK_E, 128), lambda i: (i, 0)),
          pl.BlockSpec((1, 1, BLK_E), lambda i: (i, 0, 0)),
          _full_spec(wdf_p.shape),
          _full_spec(bdf_p.shape),
          _full_spec(wfc_p.shape),
      ],
      out_specs=pl.BlockSpec((acc_rows, 32), lambda i: (0, 0)),
      out_shape=jax.ShapeDtypeStruct((acc_rows, 32), jnp.float32),
  )(distance, gj, dmi3, wdf_p, bdf_p, wfc_p)


def _mid_pass(agg0, an3, emb_p, wcf0_p, bcf0_p, bdf0_p, wfc0_p, wcf1_p,
              bcf1_p, bdf1_p, wfc1_p):
  n = agg0.shape[0]
  nb = n // BLK_N
  return pl.pallas_call(
      _mid_body,
      grid=(nb,),
      in_specs=[
          pl.BlockSpec((BLK_N, 32), lambda i: (i, 0)),
          pl.BlockSpec((BLK_N, 1), lambda i: (i, 0)),
          _full_spec(emb_p.shape),
          _full_spec(wcf0_p.shape),
          _full_spec(bcf0_p.shape),
          _full_spec(bdf0_p.shape),
          _full_spec(wfc0_p.shape),
          _full_spec(wcf1_p.shape),
          _full_spec(bcf1_p.shape),
          _full_spec(bdf1_p.shape),
          _full_spec(wfc1_p.shape),
      ],
      out_specs=[
          pl.BlockSpec((BLK_N, 128), lambda i: (i, 0)),
          pl.BlockSpec((BLK_N, 32), lambda i: (i, 0)),
      ],
      out_shape=[
          jax.ShapeDtypeStruct((n, 128), jnp.float32),
          jax.ShapeDtypeStruct((n, 32), jnp.float32),
      ],
  )(agg0, an3, emb_p, wcf0_p, bcf0_p, bdf0_p, wfc0_p, wcf1_p, bcf1_p,
    bdf1_p, wfc1_p)


def _final_pass(agg1, base1, am3, wg1_p, bg1_p, wgout_p, bgout_p, wlint_p,
                blin_p):
  n = agg1.shape[0]
  nb = n // BLK_N
  acc_rows = N_MOLS + WIN_M + 4
  return pl.pallas_call(
      _final_body,
      grid=(nb,),
      in_specs=[
          pl.BlockSpec((BLK_N, 32), lambda i: (i, 0)),
          pl.BlockSpec((BLK_N, 32), lambda i: (i, 0)),
          pl.BlockSpec((1, 1, BLK_N), lambda i: (i, 0, 0)),
          _full_spec(wg1_p.shape),
          _full_spec(bg1_p.shape),
          _full_spec(wgout_p.shape),
          _full_spec(bgout_p.shape),
          _full_spec(wlint_p.shape),
          _full_spec(blin_p.shape),
      ],
      out_specs=pl.BlockSpec((acc_rows, 16), lambda i: (0, 0)),
      out_shape=jax.ShapeDtypeStruct((acc_rows, 16), jnp.float32),
  )(agg1, base1, am3, wg1_p, bg1_p, wgout_p, bgout_p, wlint_p, blin_p)


def kernel(atom_number, distance, atom_membership, distance_membership_i,
           distance_membership_j, embedding,
           W_cf_0, b_cf_0, W_df_0, b_df_0, W_fc_0,
           W_cf_1, b_cf_1, W_df_1, b_df_1, W_fc_1,
           W_g1, b_g1, W_gout, b_gout, W_lin, b_lin):
  n = atom_number.shape[0]
  E, n_dist = distance.shape

  emb_p = _pad2(embedding, 32, 32)
  wcf0_p = _pad2(W_cf_0, 32, 64)
  bcf0_p = _pad1(b_cf_0, 64).reshape(1, 64)
  wdf0_p = _pad2(W_df_0, n_dist, 64)
  bdf0_p = _pad1(b_df_0, 64).reshape(1, 64)
  wfc0_p = _pad2(W_fc_0, 64, 32)
  wcf1_p = _pad2(W_cf_1, 32, 128)
  bcf1_p = _pad1(b_cf_1, 128).reshape(1, 128)
  wdf1_p = _pad2(W_df_1, n_dist, 64)
  bdf1_p = _pad1(b_df_1, 64).reshape(1, 64)
  bdf1_p128 = _pad1(b_df_1, 128).reshape(1, 128)
  wfc1_p = _pad2(W_fc_1, 64, 32)
  wfc1_p128 = _pad2(W_fc_1, 128, 32)
  wg1_p = _pad2(W_g1, 32, 128)
  bg1_p = _pad1(b_g1, 128).reshape(1, 128)
  wgout_p = _pad2(W_gout, 128, 16)
  bgout_p = _pad1(b_gout, 16).reshape(1, 16)
  wlint_p = _pad2(W_lin.T, 16, 16)
  blin_p = _pad1(b_lin, 16).reshape(1, 16)

  dmi3 = distance_membership_i.astype(jnp.int32).reshape(E // BLK_E, 1, BLK_E)
  an2 = atom_number.astype(jnp.int32).reshape(n, 1)
  am3 = atom_membership.astype(jnp.int32).reshape(n // BLK_N, 1, BLK_N)
  dmj = distance_membership_j.astype(jnp.int32)

  anj = _sc_gather_i32(atom_number.astype(jnp.int32), dmj)
  anj2 = anj.reshape(E, 1)

  agg0 = _edge_pass0(distance, anj2, dmi3, emb_p, wcf0_p, bcf0_p, wdf0_p,
                     bdf0_p, wfc0_p, n)[:n]
  afh1, base1 = _mid_pass(agg0, an2, emb_p, wcf0_p, bcf0_p, bdf0_p, wfc0_p,
                          wcf1_p, bcf1_p, bdf1_p128, wfc1_p128)
  gj1 = _sc_gather_rows(afh1, dmj, 128)
  agg1 = _edge_pass1(distance, gj1, dmi3, wdf1_p, bdf1_p, wfc1_p, n)[:n]
  out = _final_pass(agg1, base1, am3, wg1_p, bg1_p, wgout_p, bgout_p,
                    wlint_p, blin_p)
  return out[:N_MOLS, :12]


# speedup vs baseline: 4.1453x; 1.1239x over previous
import functools

import jax
import jax.numpy as jnp
from jax import lax
from jax.experimental import pallas as pl
from jax.experimental.pallas import tpu as pltpu
from jax.experimental.pallas import tpu_sc as plsc

N_MOLS = 2500

BLK_E = 2000
BLK_N = 2000
WIN_A = 128
WIN_M = 128


def _pad2(x, r, c):
  return jnp.pad(x, ((0, r - x.shape[0]), (0, c - x.shape[1])))


def _pad1(x, n):
  return jnp.pad(x, (0, n - x.shape[0]))


def _sc_gather_i32(table, idx):
  E = idx.shape[0]
  mesh = plsc.VectorSubcoreMesh(core_axis_name="c", subcore_axis_name="s")
  nw = mesh.num_cores * mesh.num_subcores
  per = E // nw
  C = 5000
  assert per % C == 0 and per % 8 == 0

  @functools.partial(
      pl.kernel,
      out_type=jax.ShapeDtypeStruct((E,), jnp.int32),
      mesh=mesh,
      scratch_types=[
          pltpu.VMEM((C,), jnp.int32),
          pltpu.VMEM((C,), jnp.int32),
          pltpu.SemaphoreType.DMA,
      ],
  )
  def k(table_hbm, idx_hbm, out_hbm, idx_v, rows_v, sem):
    wid = lax.axis_index("s") * mesh.num_cores + lax.axis_index("c")
    base = wid * per

    def body(i, carry):
      off = base + i * C
      pltpu.sync_copy(idx_hbm.at[pl.ds(off, C)], idx_v)
      pltpu.async_copy(table_hbm.at[idx_v], rows_v, sem).wait()
      pltpu.sync_copy(rows_v, out_hbm.at[pl.ds(off, C)])
      return carry

    lax.fori_loop(0, per // C, body, 0)

  return k(table, idx)


def _sc_gather_rows(table, idx, d):
  E = idx.shape[0]
  mesh = plsc.VectorSubcoreMesh(core_axis_name="c", subcore_axis_name="s")
  nw = mesh.num_cores * mesh.num_subcores
  per = E // nw
  C = 200
  assert per % C == 0 and C % 8 == 0 and per % 8 == 0

  @functools.partial(
      pl.kernel,
      out_type=jax.ShapeDtypeStruct((E, d), jnp.float32),
      mesh=mesh,
      scratch_types=[
          pltpu.VMEM((C,), jnp.int32),
          pltpu.VMEM((C, d), jnp.float32),
          pltpu.SemaphoreType.DMA,
      ],
  )
  def k(table_hbm, idx_hbm, out_hbm, idx_v, rows_v, sem):
    wid = lax.axis_index("s") * mesh.num_cores + lax.axis_index("c")
    base = wid * per

    def body(i, carry):
      off = base + i * C
      pltpu.sync_copy(idx_hbm.at[pl.ds(off, C)], idx_v)
      pltpu.async_copy(table_hbm.at[idx_v], rows_v, sem).wait()
      pltpu.sync_copy(rows_v, out_hbm.at[pl.ds(off, C)])
      return carry

    lax.fori_loop(0, per // C, body, 0)

  return k(table, idx)


def _window_scatter(acc_ref, ids2, vals, win):
  lo = jnp.min(ids2)
  hi = jnp.max(ids2)
  aw0 = (lo // 8) * 8
  n_win = (hi - aw0) // win + 1

  def body(w, carry):
    aw = aw0 + w * win
    rows = lax.broadcasted_iota(jnp.int32, (win, ids2.shape[1]), 0) + aw
    m = (rows == ids2).astype(vals.dtype)
    contrib = jnp.dot(m, vals, preferred_element_type=jnp.float32)
    acc_ref[pl.ds(aw, win), :] += contrib
    return carry

  lax.fori_loop(0, n_win, body, 0)


def _edge0_body(dist_ref, anj_ref, dmi_ref, emb_ref, wcf_ref, bcf_ref,
                wdf_ref, bdf_ref, wfc_ref, acc_ref):
  @pl.when(pl.program_id(0) == 0)
  def _init():
    acc_ref[...] = jnp.zeros_like(acc_ref)

  afh_t = (
      jnp.dot(emb_ref[...], wcf_ref[...], preferred_element_type=jnp.float32)
      + bcf_ref[...]
  )
  onehot_t = (
      lax.broadcasted_iota(jnp.int32, (32, BLK_E), 0) == anj_ref[0]
  ).astype(jnp.float32)
  gj = lax.dot_general(
      onehot_t, afh_t, (((0,), (0,)), ((), ())),
      preferred_element_type=jnp.float32,
  )
  dh = (
      jnp.dot(dist_ref[...], wdf_ref[...], preferred_element_type=jnp.float32)
      + bdf_ref[...]
  )
  m = (dh * gj).astype(jnp.bfloat16)
  out = jnp.tanh(
      jnp.dot(m, wfc_ref[...], preferred_element_type=jnp.float32)
  )
  _window_scatter(acc_ref, dmi_ref[0], out.astype(jnp.bfloat16), WIN_A)


def _edge1_body(dist_ref, gj_ref, dmi_ref, wdf_ref, bdf_ref, wfc_ref,
                acc_ref):
  @pl.when(pl.program_id(0) == 0)
  def _init():
    acc_ref[...] = jnp.zeros_like(acc_ref)

  dh = (
      jnp.dot(dist_ref[...], wdf_ref[...], preferred_element_type=jnp.float32)
      + bdf_ref[...]
  )
  gj = gj_ref[...][:, :64]
  m = (dh * gj).astype(jnp.bfloat16)
  out = jnp.tanh(
      jnp.dot(m, wfc_ref[...], preferred_element_type=jnp.float32)
  )
  _window_scatter(acc_ref, dmi_ref[0], out.astype(jnp.bfloat16), WIN_A)


def _mid_body(agg_ref, an_ref, emb_ref, wcf0_ref, bcf0_ref, bdf0_ref,
              wfc0_ref, wcf1_ref, bcf1_ref, bdf1_ref, wfc1_ref,
              afh1_ref, base1_ref):
  afh0_t = (
      jnp.dot(emb_ref[...], wcf0_ref[...], preferred_element_type=jnp.float32)
      + bcf0_ref[...]
  )
  out_ii0_t = jnp.tanh(
      jnp.dot(
          bdf0_ref[...] * afh0_t, wfc0_ref[...],
          preferred_element_type=jnp.float32,
      )
  )
  delta_t = emb_ref[...] - out_ii0_t
  onehot_t = (
      lax.broadcasted_iota(jnp.int32, (32, BLK_N), 0) == an_ref[0]
  ).astype(jnp.float32)
  af1 = agg_ref[...] + lax.dot_general(
      onehot_t, delta_t, (((0,), (0,)), ((), ())),
      preferred_element_type=jnp.float32,
  )
  afh1 = (
      jnp.dot(af1, wcf1_ref[...], preferred_element_type=jnp.float32)
      + bcf1_ref[...]
  )
  afh1_ref[...] = afh1
  out_ii1 = jnp.tanh(
      jnp.dot(
          bdf1_ref[...] * afh1, wfc1_ref[...],
          preferred_element_type=jnp.float32,
      )
  )
  base1_ref[...] = af1 - out_ii1


def _final_body(agg_ref, base_ref, am_ref, wg1_ref, bg1_ref, wgout_ref,
                bgout_ref, wlint_ref, blin_ref, acc_ref):
  @pl.when(pl.program_id(0) == 0)
  def _init():
    acc_ref[...] = jnp.zeros_like(acc_ref)

  af2 = agg_ref[...] + base_ref[...]
  g = jnp.tanh(
      jnp.dot(af2, wg1_ref[...], preferred_element_type=jnp.float32)
      + bg1_ref[...]
  )
  g2 = jnp.tanh(
      jnp.dot(g, wgout_ref[...], preferred_element_type=jnp.float32)
      + bgout_ref[...]
  )
  h = jnp.dot(g2, wlint_ref[...], preferred_element_type=jnp.float32)
  _window_scatter(acc_ref, am_ref[0], h, WIN_M)

  @pl.when(pl.program_id(0) == pl.num_programs(0) - 1)
  def _bias():
    acc_ref[...] += blin_ref[...]


def _full_spec(shape):
  return pl.BlockSpec(shape, lambda i: tuple(0 for _ in shape))


def _edge_pass0(distance, anj3, dmi3, emb_p, wcf_p, bcf_p, wdf_p, bdf_p,
                wfc_p, n_atoms):
  E = distance.shape[0]
  nb = E // BLK_E
  acc_rows = n_atoms + WIN_A
  return pl.pallas_call(
      _edge0_body,
      grid=(nb,),
      in_specs=[
          pl.BlockSpec((BLK_E, distance.shape[1]), lambda i: (i, 0)),
          pl.BlockSpec((1, 1, BLK_E), lambda i: (i, 0, 0)),
          pl.BlockSpec((1, 1, BLK_E), lambda i: (i, 0, 0)),
          _full_spec(emb_p.shape),
          _full_spec(wcf_p.shape),
          _full_spec(bcf_p.shape),
          _full_spec(wdf_p.shape),
          _full_spec(bdf_p.shape),
          _full_spec(wfc_p.shape),
      ],
      out_specs=pl.BlockSpec((acc_rows, 32), lambda i: (0, 0)),
      out_shape=jax.ShapeDtypeStruct((acc_rows, 32), jnp.float32),
  )(distance, anj3, dmi3, emb_p, wcf_p, bcf_p, wdf_p, bdf_p, wfc_p)


def _edge_pass1(distance, gj, dmi3, wdf_p, bdf_p, wfc_p, n_atoms):
  E = distance.shape[0]
  nb = E // BLK_E
  acc_rows = n_atoms + WIN_A
  return pl.pallas_call(
      _edge1_body,
      grid=(nb,),
      in_specs=[
          pl.BlockSpec((BLK_E, distance.shape[1]), lambda i: (i, 0)),
          pl.BlockSpec((BLK_E, 128), lambda i: (i, 0)),
          pl.BlockSpec((1, 1, BLK_E), lambda i: (i, 0, 0)),
          _full_spec(wdf_p.shape),
          _full_spec(bdf_p.shape),
          _full_spec(wfc_p.shape),
      ],
      out_specs=pl.BlockSpec((acc_rows, 32), lambda i: (0, 0)),
      out_shape=jax.ShapeDtypeStruct((acc_rows, 32), jnp.float32),
  )(distance, gj, dmi3, wdf_p, bdf_p, wfc_p)


def _mid_pass(agg0, an3, emb_p, wcf0_p, bcf0_p, bdf0_p, wfc0_p, wcf1_p,
              bcf1_p, bdf1_p, wfc1_p):
  n = agg0.shape[0]
  nb = n // BLK_N
  return pl.pallas_call(
      _mid_body,
      grid=(nb,),
      in_specs=[
          pl.BlockSpec((BLK_N, 32), lambda i: (i, 0)),
          pl.BlockSpec((1, 1, BLK_N), lambda i: (i, 0, 0)),
          _full_spec(emb_p.shape),
          _full_spec(wcf0_p.shape),
          _full_spec(bcf0_p.shape),
          _full_spec(bdf0_p.shape),
          _full_spec(wfc0_p.shape),
          _full_spec(wcf1_p.shape),
          _full_spec(bcf1_p.shape),
          _full_spec(bdf1_p.shape),
          _full_spec(wfc1_p.shape),
      ],
      out_specs=[
          pl.BlockSpec((BLK_N, 128), lambda i: (i, 0)),
          pl.BlockSpec((BLK_N, 32), lambda i: (i, 0)),
      ],
      out_shape=[
          jax.ShapeDtypeStruct((n, 128), jnp.float32),
          jax.ShapeDtypeStruct((n, 32), jnp.float32),
      ],
  )(agg0, an3, emb_p, wcf0_p, bcf0_p, bdf0_p, wfc0_p, wcf1_p, bcf1_p,
    bdf1_p, wfc1_p)


def _final_pass(agg1, base1, am3, wg1_p, bg1_p, wgout_p, bgout_p, wlint_p,
                blin_p):
  n = agg1.shape[0]
  nb = n // BLK_N
  acc_rows = N_MOLS + WIN_M + 4
  return pl.pallas_call(
      _final_body,
      grid=(nb,),
      in_specs=[
          pl.BlockSpec((BLK_N, 32), lambda i: (i, 0)),
          pl.BlockSpec((BLK_N, 32), lambda i: (i, 0)),
          pl.BlockSpec((1, 1, BLK_N), lambda i: (i, 0, 0)),
          _full_spec(wg1_p.shape),
          _full_spec(bg1_p.shape),
          _full_spec(wgout_p.shape),
          _full_spec(bgout_p.shape),
          _full_spec(wlint_p.shape),
          _full_spec(blin_p.shape),
      ],
      out_specs=pl.BlockSpec((acc_rows, 16), lambda i: (0, 0)),
      out_shape=jax.ShapeDtypeStruct((acc_rows, 16), jnp.float32),
  )(agg1, base1, am3, wg1_p, bg1_p, wgout_p, bgout_p, wlint_p, blin_p)


def kernel(atom_number, distance, atom_membership, distance_membership_i,
           distance_membership_j, embedding,
           W_cf_0, b_cf_0, W_df_0, b_df_0, W_fc_0,
           W_cf_1, b_cf_1, W_df_1, b_df_1, W_fc_1,
           W_g1, b_g1, W_gout, b_gout, W_lin, b_lin):
  n = atom_number.shape[0]
  E, n_dist = distance.shape

  emb_p = _pad2(embedding, 32, 32)
  wcf0_p = _pad2(W_cf_0, 32, 64)
  bcf0_p = _pad1(b_cf_0, 64).reshape(1, 64)
  wdf0_p = _pad2(W_df_0, n_dist, 64)
  wdf0_b = wdf0_p.astype(jnp.bfloat16)
  bdf0_p = _pad1(b_df_0, 64).reshape(1, 64)
  wfc0_p = _pad2(W_fc_0, 64, 32)
  wfc0_b = wfc0_p.astype(jnp.bfloat16)
  wcf1_p = _pad2(W_cf_1, 32, 128)
  bcf1_p = _pad1(b_cf_1, 128).reshape(1, 128)
  wdf1_p = _pad2(W_df_1, n_dist, 64)
  wdf1_b = wdf1_p.astype(jnp.bfloat16)
  bdf1_p = _pad1(b_df_1, 64).reshape(1, 64)
  bdf1_p128 = _pad1(b_df_1, 128).reshape(1, 128)
  wfc1_p = _pad2(W_fc_1, 64, 32)
  wfc1_b = wfc1_p.astype(jnp.bfloat16)
  wfc1_p128 = _pad2(W_fc_1, 128, 32)
  wg1_p = _pad2(W_g1, 32, 128)
  bg1_p = _pad1(b_g1, 128).reshape(1, 128)
  wgout_p = _pad2(W_gout, 128, 16)
  bgout_p = _pad1(b_gout, 16).reshape(1, 16)
  wlint_p = _pad2(W_lin.T, 16, 16)
  blin_p = _pad1(b_lin, 16).reshape(1, 16)

  dmi3 = distance_membership_i.astype(jnp.int32).reshape(E // BLK_E, 1, BLK_E)
  an3 = atom_number.astype(jnp.int32).reshape(n // BLK_N, 1, BLK_N)
  am3 = atom_membership.astype(jnp.int32).reshape(n // BLK_N, 1, BLK_N)
  dmj = distance_membership_j.astype(jnp.int32)
  dist_b = distance.astype(jnp.bfloat16)

  anj = _sc_gather_i32(atom_number.astype(jnp.int32), dmj)
  anj3 = anj.reshape(E // BLK_E, 1, BLK_E)

  agg0 = _edge_pass0(dist_b, anj3, dmi3, emb_p, wcf0_p, bcf0_p, wdf0_b,
                     bdf0_p, wfc0_b, n)[:n]
  afh1, base1 = _mid_pass(agg0, an3, emb_p, wcf0_p, bcf0_p, bdf0_p, wfc0_p,
                          wcf1_p, bcf1_p, bdf1_p128, wfc1_p128)
  gj1 = _sc_gather_rows(afh1, dmj, 128)
  agg1 = _edge_pass1(dist_b, gj1, dmi3, wdf1_b, bdf1_p, wfc1_b, n)[:n]
  out = _final_pass(agg1, base1, am3, wg1_p, bg1_p, wgout_p, bgout_p,
                    wlint_p, blin_p)
  return out[:N_MOLS, :12]


# speedup vs baseline: 4.2993x; 1.0372x over previous
import functools

import jax
import jax.numpy as jnp
from jax import lax
from jax.experimental import pallas as pl
from jax.experimental.pallas import tpu as pltpu
from jax.experimental.pallas import tpu_sc as plsc

N_MOLS = 2500

BLK_E = 2000
BLK_N = 2000
WIN_A = 128
WIN_M = 128


def _pad2(x, r, c):
  return jnp.pad(x, ((0, r - x.shape[0]), (0, c - x.shape[1])))


def _pad1(x, n):
  return jnp.pad(x, (0, n - x.shape[0]))


def _sc_gather(table, idx, C):
  E = idx.shape[0]
  row = table.shape[1:]
  mesh = plsc.VectorSubcoreMesh(core_axis_name="c", subcore_axis_name="s")
  nw = mesh.num_cores * mesh.num_subcores
  per = E // nw
  n = per // C
  assert per % C == 0 and C % 8 == 0 and n % 2 == 1 and n >= 5

  @functools.partial(
      pl.kernel,
      out_type=jax.ShapeDtypeStruct((E,) + row, table.dtype),
      mesh=mesh,
      scratch_types=[
          pltpu.VMEM((C,), jnp.int32),
          pltpu.VMEM((C,), jnp.int32),
          pltpu.VMEM((C,) + row, table.dtype),
          pltpu.VMEM((C,) + row, table.dtype),
          pltpu.SemaphoreType.DMA,
          pltpu.SemaphoreType.DMA,
          pltpu.SemaphoreType.DMA,
          pltpu.SemaphoreType.DMA,
      ],
  )
  def k(table_hbm, idx_hbm, out_hbm, idx0, idx1, rows0, rows1,
        sg0, sg1, sw0, sw1):
    wid = lax.axis_index("s") * mesh.num_cores + lax.axis_index("c")
    base = wid * per
    idx_v = (idx0, idx1)
    rows_v = (rows0, rows1)
    sg = (sg0, sg1)
    sw = (sw0, sw1)

    def wait_gather(b):
      pltpu.make_async_copy(table_hbm.at[idx_v[b]], rows_v[b], sg[b]).wait()

    def wait_writeout(b):
      pltpu.make_async_copy(
          rows_v[b], out_hbm.at[pl.ds(base, C)], sw[b]).wait()

    def start(i, b, first):
      if not first:
        wait_writeout(b)
      pltpu.sync_copy(idx_hbm.at[pl.ds(base + i * C, C)], idx_v[b])
      pltpu.async_copy(table_hbm.at[idx_v[b]], rows_v[b], sg[b])

    def drain(i, b):
      wait_gather(b)
      pltpu.async_copy(rows_v[b], out_hbm.at[pl.ds(base + i * C, C)], sw[b])

    start(0, 0, True)
    start(1, 1, True)
    drain(0, 0)
    start(2, 0, False)

    def pair(p, carry):
      i = 3 + 2 * p
      drain(i - 2, 1)
      start(i, 1, False)
      drain(i - 1, 0)
      start(i + 1, 0, False)
      return carry

    lax.fori_loop(0, (n - 3) // 2, pair, 0)
    drain(n - 2, 1)
    drain(n - 1, 0)
    wait_writeout(1)
    wait_writeout(0)

  return k(table, idx)


def _window_scatter(acc_ref, ids2, vals, win):
  lo = jnp.min(ids2)
  hi = jnp.max(ids2)
  aw0 = (lo // 8) * 8
  n_win = (hi - aw0) // win + 1

  def body(w, carry):
    aw = aw0 + w * win
    rows = lax.broadcasted_iota(jnp.int32, (win, ids2.shape[1]), 0) + aw
    m = (rows == ids2).astype(vals.dtype)
    contrib = jnp.dot(m, vals, preferred_element_type=jnp.float32)
    acc_ref[pl.ds(aw, win), :] += contrib
    return carry

  lax.fori_loop(0, n_win, body, 0)


def _edge0_body(dist_ref, anj_ref, dmi_ref, emb_ref, wcf_ref, bcf_ref,
                wdf_ref, bdf_ref, wfc_ref, acc_ref, distb_ref):
  @pl.when(pl.program_id(0) == 0)
  def _init():
    acc_ref[...] = jnp.zeros_like(acc_ref)

  afh_t = (
      jnp.dot(emb_ref[...], wcf_ref[...], preferred_element_type=jnp.float32)
      + bcf_ref[...]
  )
  onehot_t = (
      lax.broadcasted_iota(jnp.int32, (32, BLK_E), 0) == anj_ref[0]
  ).astype(jnp.float32)
  gj = lax.dot_general(
      onehot_t, afh_t, (((0,), (0,)), ((), ())),
      preferred_element_type=jnp.float32,
  )
  dist_b = dist_ref[...].astype(jnp.bfloat16)
  distb_ref[...] = dist_b
  dh = (
      jnp.dot(dist_b, wdf_ref[...], preferred_element_type=jnp.float32)
      + bdf_ref[...]
  )
  m = (dh * gj).astype(jnp.bfloat16)
  out = jnp.tanh(
      jnp.dot(m, wfc_ref[...], preferred_element_type=jnp.float32)
  )
  _window_scatter(acc_ref, dmi_ref[0], out.astype(jnp.bfloat16), WIN_A)


def _edge1_body(dist_ref, gj_ref, dmi_ref, wdf_ref, bdf_ref, wfc_ref,
                acc_ref):
  @pl.when(pl.program_id(0) == 0)
  def _init():
    acc_ref[...] = jnp.zeros_like(acc_ref)

  dh = (
      jnp.dot(dist_ref[...], wdf_ref[...], preferred_element_type=jnp.float32)
      + bdf_ref[...]
  )
  gj = gj_ref[...][:, :64]
  m = (dh * gj).astype(jnp.bfloat16)
  out = jnp.tanh(
      jnp.dot(m, wfc_ref[...], preferred_element_type=jnp.float32)
  )
  _window_scatter(acc_ref, dmi_ref[0], out.astype(jnp.bfloat16), WIN_A)


def _mid_body(agg_ref, an_ref, emb_ref, wcf0_ref, bcf0_ref, bdf0_ref,
              wfc0_ref, wcf1_ref, bcf1_ref, bdf1_ref, wfc1_ref,
              afh1_ref, base1_ref):
  afh0_t = (
      jnp.dot(emb_ref[...], wcf0_ref[...], preferred_element_type=jnp.float32)
      + bcf0_ref[...]
  )
  out_ii0_t = jnp.tanh(
      jnp.dot(
          bdf0_ref[...] * afh0_t, wfc0_ref[...],
          preferred_element_type=jnp.float32,
      )
  )
  delta_t = emb_ref[...] - out_ii0_t
  onehot_t = (
      lax.broadcasted_iota(jnp.int32, (32, BLK_N), 0) == an_ref[0]
  ).astype(jnp.float32)
  af1 = agg_ref[...] + lax.dot_general(
      onehot_t, delta_t, (((0,), (0,)), ((), ())),
      preferred_element_type=jnp.float32,
  )
  afh1 = (
      jnp.dot(af1, wcf1_ref[...], preferred_element_type=jnp.float32)
      + bcf1_ref[...]
  )
  afh1_ref[...] = afh1
  out_ii1 = jnp.tanh(
      jnp.dot(
          bdf1_ref[...] * afh1, wfc1_ref[...],
          preferred_element_type=jnp.float32,
      )
  )
  base1_ref[...] = af1 - out_ii1


def _final_body(agg_ref, base_ref, am_ref, wg1_ref, bg1_ref, wgout_ref,
                bgout_ref, wlint_ref, blin_ref, acc_ref):
  @pl.when(pl.program_id(0) == 0)
  def _init():
    acc_ref[...] = jnp.zeros_like(acc_ref)

  af2 = agg_ref[...] + base_ref[...]
  g = jnp.tanh(
      jnp.dot(af2, wg1_ref[...], preferred_element_type=jnp.float32)
      + bg1_ref[...]
  )
  g2 = jnp.tanh(
      jnp.dot(g, wgout_ref[...], preferred_element_type=jnp.float32)
      + bgout_ref[...]
  )
  h = jnp.dot(g2, wlint_ref[...], preferred_element_type=jnp.float32)
  _window_scatter(acc_ref, am_ref[0], h, WIN_M)

  @pl.when(pl.program_id(0) == pl.num_programs(0) - 1)
  def _bias():
    acc_ref[...] += blin_ref[...]


def _full_spec(shape):
  return pl.BlockSpec(shape, lambda i: tuple(0 for _ in shape))


def _edge_pass0(distance, anj3, dmi3, emb_p, wcf_p, bcf_p, wdf_p, bdf_p,
                wfc_p, n_atoms):
  E = distance.shape[0]
  nb = E // BLK_E
  acc_rows = n_atoms + WIN_A
  return pl.pallas_call(
      _edge0_body,
      grid=(nb,),
      in_specs=[
          pl.BlockSpec((BLK_E, distance.shape[1]), lambda i: (i, 0)),
          pl.BlockSpec((1, 1, BLK_E), lambda i: (i, 0, 0)),
          pl.BlockSpec((1, 1, BLK_E), lambda i: (i, 0, 0)),
          _full_spec(emb_p.shape),
          _full_spec(wcf_p.shape),
          _full_spec(bcf_p.shape),
          _full_spec(wdf_p.shape),
          _full_spec(bdf_p.shape),
          _full_spec(wfc_p.shape),
      ],
      out_specs=[
          pl.BlockSpec((acc_rows, 32), lambda i: (0, 0)),
          pl.BlockSpec((BLK_E, distance.shape[1]), lambda i: (i, 0)),
      ],
      out_shape=[
          jax.ShapeDtypeStruct((acc_rows, 32), jnp.float32),
          jax.ShapeDtypeStruct((E, distance.shape[1]), jnp.bfloat16),
      ],
  )(distance, anj3, dmi3, emb_p, wcf_p, bcf_p, wdf_p, bdf_p, wfc_p)


def _edge_pass1(distance, gj, dmi3, wdf_p, bdf_p, wfc_p, n_atoms):
  E = distance.shape[0]
  nb = E // BLK_E
  acc_rows = n_atoms + WIN_A
  return pl.pallas_call(
      _edge1_body,
      grid=(nb,),
      in_specs=[
          pl.BlockSpec((BLK_E, distance.shape[1]), lambda i: (i, 0)),
          pl.BlockSpec((BLK_E, 128), lambda i: (i, 0)),
          pl.BlockSpec((1, 1, BLK_E), lambda i: (i, 0, 0)),
          _full_spec(wdf_p.shape),
          _full_spec(bdf_p.shape),
          _full_spec(wfc_p.shape),
      ],
      out_specs=pl.BlockSpec((acc_rows, 32), lambda i: (0, 0)),
      out_shape=jax.ShapeDtypeStruct((acc_rows, 32), jnp.float32),
  )(distance, gj, dmi3, wdf_p, bdf_p, wfc_p)


def _mid_pass(agg0, an3, emb_p, wcf0_p, bcf0_p, bdf0_p, wfc0_p, wcf1_p,
              bcf1_p, bdf1_p, wfc1_p):
  n = agg0.shape[0]
  nb = n // BLK_N
  return pl.pallas_call(
      _mid_body,
      grid=(nb,),
      in_specs=[
          pl.BlockSpec((BLK_N, 32), lambda i: (i, 0)),
          pl.BlockSpec((1, 1, BLK_N), lambda i: (i, 0, 0)),
          _full_spec(emb_p.shape),
          _full_spec(wcf0_p.shape),
          _full_spec(bcf0_p.shape),
          _full_spec(bdf0_p.shape),
          _full_spec(wfc0_p.shape),
          _full_spec(wcf1_p.shape),
          _full_spec(bcf1_p.shape),
          _full_spec(bdf1_p.shape),
          _full_spec(wfc1_p.shape),
      ],
      out_specs=[
          pl.BlockSpec((BLK_N, 128), lambda i: (i, 0)),
          pl.BlockSpec((BLK_N, 32), lambda i: (i, 0)),
      ],
      out_shape=[
          jax.ShapeDtypeStruct((n, 128), jnp.float32),
          jax.ShapeDtypeStruct((n, 32), jnp.float32),
      ],
  )(agg0, an3, emb_p, wcf0_p, bcf0_p, bdf0_p, wfc0_p, wcf1_p, bcf1_p,
    bdf1_p, wfc1_p)


def _final_pass(agg1, base1, am3, wg1_p, bg1_p, wgout_p, bgout_p, wlint_p,
                blin_p):
  n = agg1.shape[0]
  nb = n // BLK_N
  acc_rows = N_MOLS + WIN_M + 4
  return pl.pallas_call(
      _final_body,
      grid=(nb,),
      in_specs=[
          pl.BlockSpec((BLK_N, 32), lambda i: (i, 0)),
          pl.BlockSpec((BLK_N, 32), lambda i: (i, 0)),
          pl.BlockSpec((1, 1, BLK_N), lambda i: (i, 0, 0)),
          _full_spec(wg1_p.shape),
          _full_spec(bg1_p.shape),
          _full_spec(wgout_p.shape),
          _full_spec(bgout_p.shape),
          _full_spec(wlint_p.shape),
          _full_spec(blin_p.shape),
      ],
      out_specs=pl.BlockSpec((acc_rows, 16), lambda i: (0, 0)),
      out_shape=jax.ShapeDtypeStruct((acc_rows, 16), jnp.float32),
  )(agg1, base1, am3, wg1_p, bg1_p, wgout_p, bgout_p, wlint_p, blin_p)


def kernel(atom_number, distance, atom_membership, distance_membership_i,
           distance_membership_j, embedding,
           W_cf_0, b_cf_0, W_df_0, b_df_0, W_fc_0,
           W_cf_1, b_cf_1, W_df_1, b_df_1, W_fc_1,
           W_g1, b_g1, W_gout, b_gout, W_lin, b_lin):
  n = atom_number.shape[0]
  E, n_dist = distance.shape

  emb_p = _pad2(embedding, 32, 32)
  wcf0_p = _pad2(W_cf_0, 32, 64)
  bcf0_p = _pad1(b_cf_0, 64).reshape(1, 64)
  wdf0_p = _pad2(W_df_0, n_dist, 64)
  wdf0_b = wdf0_p.astype(jnp.bfloat16)
  bdf0_p = _pad1(b_df_0, 64).reshape(1, 64)
  wfc0_p = _pad2(W_fc_0, 64, 32)
  wfc0_b = wfc0_p.astype(jnp.bfloat16)
  wcf1_p = _pad2(W_cf_1, 32, 128)
  bcf1_p = _pad1(b_cf_1, 128).reshape(1, 128)
  wdf1_p = _pad2(W_df_1, n_dist, 64)
  wdf1_b = wdf1_p.astype(jnp.bfloat16)
  bdf1_p = _pad1(b_df_1, 64).reshape(1, 64)
  bdf1_p128 = _pad1(b_df_1, 128).reshape(1, 128)
  wfc1_p = _pad2(W_fc_1, 64, 32)
  wfc1_b = wfc1_p.astype(jnp.bfloat16)
  wfc1_p128 = _pad2(W_fc_1, 128, 32)
  wg1_p = _pad2(W_g1, 32, 128)
  bg1_p = _pad1(b_g1, 128).reshape(1, 128)
  wgout_p = _pad2(W_gout, 128, 16)
  bgout_p = _pad1(b_gout, 16).reshape(1, 16)
  wlint_p = _pad2(W_lin.T, 16, 16)
  blin_p = _pad1(b_lin, 16).reshape(1, 16)

  dmi3 = distance_membership_i.astype(jnp.int32).reshape(E // BLK_E, 1, BLK_E)
  an3 = atom_number.astype(jnp.int32).reshape(n // BLK_N, 1, BLK_N)
  am3 = atom_membership.astype(jnp.int32).reshape(n // BLK_N, 1, BLK_N)
  dmj = distance_membership_j.astype(jnp.int32)

  anj = _sc_gather(atom_number.astype(jnp.int32), dmj, 5000)
  anj3 = anj.reshape(E // BLK_E, 1, BLK_E)

  agg0, dist_b = _edge_pass0(distance, anj3, dmi3, emb_p, wcf0_p, bcf0_p,
                             wdf0_b, bdf0_p, wfc0_b, n)
  agg0 = agg0[:n]
  afh1, base1 = _mid_pass(agg0, an3, emb_p, wcf0_p, bcf0_p, bdf0_p, wfc0_p,
                          wcf1_p, bcf1_p, bdf1_p128, wfc1_p128)
  gj1 = _sc_gather(afh1, dmj, 200)
  agg1 = _edge_pass1(dist_b, gj1, dmi3, wdf1_b, bdf1_p, wfc1_b, n)[:n]
  out = _final_pass(agg1, base1, am3, wg1_p, bg1_p, wgout_p, bgout_p,
                    wlint_p, blin_p)
  return out[:N_MOLS, :12]


# speedup vs baseline: 6.8286x; 1.5883x over previous
import functools

import jax
import jax.numpy as jnp
from jax import lax
from jax.experimental import pallas as pl
from jax.experimental.pallas import tpu as pltpu
from jax.experimental.pallas import tpu_sc as plsc

N_MOLS = 2500

BLK_E = 3200
BLK_N = 5000
WIN_A = 256
WIN_M = 128


def _pad2(x, r, c):
  return jnp.pad(x, ((0, r - x.shape[0]), (0, c - x.shape[1])))


def _pad1(x, n):
  return jnp.pad(x, (0, n - x.shape[0]))


def _sc_gather(table, idx, C):
  E = idx.shape[0]
  row = table.shape[1:]
  mesh = plsc.VectorSubcoreMesh(core_axis_name="c", subcore_axis_name="s")
  nw = mesh.num_cores * mesh.num_subcores
  per = E // nw
  n = per // C
  assert per % C == 0 and C % 8 == 0 and n % 2 == 1 and n >= 5

  @functools.partial(
      pl.kernel,
      out_type=jax.ShapeDtypeStruct((E,) + row, table.dtype),
      mesh=mesh,
      scratch_types=[
          pltpu.VMEM((C,), jnp.int32),
          pltpu.VMEM((C,), jnp.int32),
          pltpu.VMEM((C,) + row, table.dtype),
          pltpu.VMEM((C,) + row, table.dtype),
          pltpu.SemaphoreType.DMA,
          pltpu.SemaphoreType.DMA,
          pltpu.SemaphoreType.DMA,
          pltpu.SemaphoreType.DMA,
      ],
  )
  def k(table_hbm, idx_hbm, out_hbm, idx0, idx1, rows0, rows1,
        sg0, sg1, sw0, sw1):
    wid = lax.axis_index("s") * mesh.num_cores + lax.axis_index("c")
    base = wid * per
    idx_v = (idx0, idx1)
    rows_v = (rows0, rows1)
    sg = (sg0, sg1)
    sw = (sw0, sw1)

    def wait_gather(b):
      pltpu.make_async_copy(table_hbm.at[idx_v[b]], rows_v[b], sg[b]).wait()

    def wait_writeout(b):
      pltpu.make_async_copy(
          rows_v[b], out_hbm.at[pl.ds(base, C)], sw[b]).wait()

    def start(i, b, first):
      if not first:
        wait_writeout(b)
      pltpu.sync_copy(idx_hbm.at[pl.ds(base + i * C, C)], idx_v[b])
      pltpu.async_copy(table_hbm.at[idx_v[b]], rows_v[b], sg[b])

    def drain(i, b):
      wait_gather(b)
      pltpu.async_copy(rows_v[b], out_hbm.at[pl.ds(base + i * C, C)], sw[b])

    start(0, 0, True)
    start(1, 1, True)
    drain(0, 0)
    start(2, 0, False)

    def pair(p, carry):
      i = 3 + 2 * p
      drain(i - 2, 1)
      start(i, 1, False)
      drain(i - 1, 0)
      start(i + 1, 0, False)
      return carry

    lax.fori_loop(0, (n - 3) // 2, pair, 0)
    drain(n - 2, 1)
    drain(n - 1, 0)
    wait_writeout(1)
    wait_writeout(0)

  return k(table, idx)


def _window_scatter(acc_ref, aw0, n_win, ids2, vals, win):
  rows = lax.broadcasted_iota(jnp.int32, (win, ids2.shape[1]), 0)

  def body(w, carry):
    aw = aw0 + w * win
    m = (rows == ids2 - aw).astype(vals.dtype)
    contrib = jnp.dot(m, vals, preferred_element_type=jnp.float32)
    acc_ref[pl.ds(aw, win), :] += contrib
    return carry

  lax.fori_loop(0, n_win, body, 0)


def _edge0_body(dist_ref, anj_ref, dmi_ref, aw_ref, nw_ref, emb_ref,
                wcf_ref, bcf_ref, wdf_ref, bdf_ref, wfc_ref, acc_ref,
                distb_ref):
  @pl.when(pl.program_id(0) == 0)
  def _init():
    acc_ref[...] = jnp.zeros_like(acc_ref)

  afh_t = (
      jnp.dot(emb_ref[...], wcf_ref[...], preferred_element_type=jnp.float32)
      + bcf_ref[...]
  )
  onehot_t = (
      lax.broadcasted_iota(jnp.int32, (32, BLK_E), 0) == anj_ref[0]
  ).astype(jnp.float32)
  gj = lax.dot_general(
      onehot_t, afh_t, (((0,), (0,)), ((), ())),
      preferred_element_type=jnp.float32,
  )
  dist_b = dist_ref[...].astype(jnp.bfloat16)
  distb_ref[...] = dist_b
  dh = (
      lax.dot_general(
          dist_b, wdf_ref[...], (((0,), (0,)), ((), ())),
          preferred_element_type=jnp.float32,
      )
      + bdf_ref[...]
  )
  m = (dh * gj).astype(jnp.bfloat16)
  out = jnp.tanh(
      jnp.dot(m, wfc_ref[...], preferred_element_type=jnp.float32)
  )
  i = pl.program_id(0)
  _window_scatter(acc_ref, aw_ref[i], nw_ref[i], dmi_ref[0],
                  out.astype(jnp.bfloat16), WIN_A)


def _edge1_body(off, dist_ref, gj_ref, dmi_ref, aw_ref, nw_ref, wdf_ref,
                bdf_ref, wfc_ref, acc_ref):
  @pl.when(pl.program_id(0) == 0)
  def _init():
    acc_ref[...] = jnp.zeros_like(acc_ref)

  dh = (
      lax.dot_general(
          dist_ref[...], wdf_ref[...], (((0,), (0,)), ((), ())),
          preferred_element_type=jnp.float32,
      )
      + bdf_ref[...]
  )
  gj = gj_ref[...][:, :64]
  m = (dh * gj).astype(jnp.bfloat16)
  out = jnp.tanh(
      jnp.dot(m, wfc_ref[...], preferred_element_type=jnp.float32)
  )
  i = pl.program_id(0) + off
  _window_scatter(acc_ref, aw_ref[i], nw_ref[i], dmi_ref[0],
                  out.astype(jnp.bfloat16), WIN_A)


def _mid_body(agg_ref, an_ref, emb_ref, wcf0_ref, bcf0_ref, bdf0_ref,
              wfc0_ref, wcf1_ref, bcf1_ref, bdf1_ref, wfc1_ref,
              afh1_ref, base1_ref):
  afh0_t = (
      jnp.dot(emb_ref[...], wcf0_ref[...], preferred_element_type=jnp.float32)
      + bcf0_ref[...]
  )
  out_ii0_t = jnp.tanh(
      jnp.dot(
          bdf0_ref[...] * afh0_t, wfc0_ref[...],
          preferred_element_type=jnp.float32,
      )
  )
  delta_t = emb_ref[...] - out_ii0_t
  onehot_t = (
      lax.broadcasted_iota(jnp.int32, (32, BLK_N), 0) == an_ref[0]
  ).astype(jnp.float32)
  af1 = agg_ref[...] + lax.dot_general(
      onehot_t, delta_t, (((0,), (0,)), ((), ())),
      preferred_element_type=jnp.float32,
  )
  afh1 = (
      jnp.dot(af1, wcf1_ref[...], preferred_element_type=jnp.float32)
      + bcf1_ref[...]
  )
  afh1_ref[...] = afh1
  out_ii1 = jnp.tanh(
      jnp.dot(
          bdf1_ref[...] * afh1, wfc1_ref[...],
          preferred_element_type=jnp.float32,
      )
  )
  base1_ref[...] = af1 - out_ii1


def _final_body(agg_ref, base_ref, am_ref, aw_ref, nw_ref, wg1_ref, bg1_ref,
                wgout_ref, bgout_ref, wlint_ref, blin_ref, acc_ref):
  @pl.when(pl.program_id(0) == 0)
  def _init():
    acc_ref[...] = jnp.zeros_like(acc_ref)

  af2 = agg_ref[...] + base_ref[...]
  g = jnp.tanh(
      jnp.dot(af2, wg1_ref[...], preferred_element_type=jnp.float32)
      + bg1_ref[...]
  )
  g2 = jnp.tanh(
      jnp.dot(g, wgout_ref[...], preferred_element_type=jnp.float32)
      + bgout_ref[...]
  )
  h = jnp.dot(g2, wlint_ref[...], preferred_element_type=jnp.float32)
  i = pl.program_id(0)
  _window_scatter(acc_ref, aw_ref[i], nw_ref[i], am_ref[0], h, WIN_M)

  @pl.when(pl.program_id(0) == pl.num_programs(0) - 1)
  def _bias():
    acc_ref[...] += blin_ref[...]


def _full_spec(shape):
  return pl.BlockSpec(shape, lambda i: tuple(0 for _ in shape))


def _edge_pass0(dist_t, anj3, dmi3, aw_e, nw_e, emb_p, wcf_p, bcf_p, wdf_p,
                bdf_p, wfc_p, n_atoms):
  n_dist, E = dist_t.shape
  nb = E // BLK_E
  acc_rows = n_atoms + WIN_A
  return pl.pallas_call(
      _edge0_body,
      grid=(nb,),
      in_specs=[
          pl.BlockSpec((n_dist, BLK_E), lambda i: (0, i)),
          pl.BlockSpec((1, 1, BLK_E), lambda i: (i, 0, 0)),
          pl.BlockSpec((1, 1, BLK_E), lambda i: (i, 0, 0)),
          pl.BlockSpec(memory_space=pltpu.SMEM),
          pl.BlockSpec(memory_space=pltpu.SMEM),
          _full_spec(emb_p.shape),
          _full_spec(wcf_p.shape),
          _full_spec(bcf_p.shape),
          _full_spec(wdf_p.shape),
          _full_spec(bdf_p.shape),
          _full_spec(wfc_p.shape),
      ],
      out_specs=[
          pl.BlockSpec((acc_rows, 32), lambda i: (0, 0)),
          pl.BlockSpec((n_dist, BLK_E), lambda i: (0, i)),
      ],
      out_shape=[
          jax.ShapeDtypeStruct((acc_rows, 32), jnp.float32),
          jax.ShapeDtypeStruct((n_dist, E), jnp.bfloat16),
      ],
  )(dist_t, anj3, dmi3, aw_e, nw_e, emb_p, wcf_p, bcf_p, wdf_p, bdf_p,
    wfc_p)


def _edge_pass1(dist_t, gj, dmi3, aw_e, nw_e, wdf_p, bdf_p, wfc_p,
                n_atoms, off):
  n_dist, _ = dist_t.shape
  nb = gj.shape[0] // BLK_E
  acc_rows = n_atoms + WIN_A
  return pl.pallas_call(
      functools.partial(_edge1_body, off),
      grid=(nb,),
      in_specs=[
          pl.BlockSpec((n_dist, BLK_E), lambda i: (0, i + off)),
          pl.BlockSpec((BLK_E, 128), lambda i: (i, 0)),
          pl.BlockSpec((1, 1, BLK_E), lambda i: (i + off, 0, 0)),
          pl.BlockSpec(memory_space=pltpu.SMEM),
          pl.BlockSpec(memory_space=pltpu.SMEM),
          _full_spec(wdf_p.shape),
          _full_spec(bdf_p.shape),
          _full_spec(wfc_p.shape),
      ],
      out_specs=pl.BlockSpec((acc_rows, 32), lambda i: (0, 0)),
      out_shape=jax.ShapeDtypeStruct((acc_rows, 32), jnp.float32),
  )(dist_t, gj, dmi3, aw_e, nw_e, wdf_p, bdf_p, wfc_p)


def _mid_pass(agg0, an3, emb_p, wcf0_p, bcf0_p, bdf0_p, wfc0_p, wcf1_p,
              bcf1_p, bdf1_p, wfc1_p):
  n = agg0.shape[0]
  nb = n // BLK_N
  return pl.pallas_call(
      _mid_body,
      grid=(nb,),
      in_specs=[
          pl.BlockSpec((BLK_N, 32), lambda i: (i, 0)),
          pl.BlockSpec((1, 1, BLK_N), lambda i: (i, 0, 0)),
          _full_spec(emb_p.shape),
          _full_spec(wcf0_p.shape),
          _full_spec(bcf0_p.shape),
          _full_spec(bdf0_p.shape),
          _full_spec(wfc0_p.shape),
          _full_spec(wcf1_p.shape),
          _full_spec(bcf1_p.shape),
          _full_spec(bdf1_p.shape),
          _full_spec(wfc1_p.shape),
      ],
      out_specs=[
          pl.BlockSpec((BLK_N, 128), lambda i: (i, 0)),
          pl.BlockSpec((BLK_N, 32), lambda i: (i, 0)),
      ],
      out_shape=[
          jax.ShapeDtypeStruct((n, 128), jnp.float32),
          jax.ShapeDtypeStruct((n, 32), jnp.float32),
      ],
  )(agg0, an3, emb_p, wcf0_p, bcf0_p, bdf0_p, wfc0_p, wcf1_p, bcf1_p,
    bdf1_p, wfc1_p)


def _final_pass(agg1, base1, am3, aw_m, nw_m, wg1_p, bg1_p, wgout_p,
                bgout_p, wlint_p, blin_p):
  n = agg1.shape[0]
  nb = n // BLK_N
  acc_rows = N_MOLS + WIN_M + 4
  return pl.pallas_call(
      _final_body,
      grid=(nb,),
      in_specs=[
          pl.BlockSpec((BLK_N, 32), lambda i: (i, 0)),
          pl.BlockSpec((BLK_N, 32), lambda i: (i, 0)),
          pl.BlockSpec((1, 1, BLK_N), lambda i: (i, 0, 0)),
          pl.BlockSpec(memory_space=pltpu.SMEM),
          pl.BlockSpec(memory_space=pltpu.SMEM),
          _full_spec(wg1_p.shape),
          _full_spec(bg1_p.shape),
          _full_spec(wgout_p.shape),
          _full_spec(bgout_p.shape),
          _full_spec(wlint_p.shape),
          _full_spec(blin_p.shape),
      ],
      out_specs=pl.BlockSpec((acc_rows, 16), lambda i: (0, 0)),
      out_shape=jax.ShapeDtypeStruct((acc_rows, 16), jnp.float32),
  )(agg1, base1, am3, aw_m, nw_m, wg1_p, bg1_p, wgout_p, bgout_p, wlint_p,
    blin_p)


def kernel(atom_number, distance, atom_membership, distance_membership_i,
           distance_membership_j, embedding,
           W_cf_0, b_cf_0, W_df_0, b_df_0, W_fc_0,
           W_cf_1, b_cf_1, W_df_1, b_df_1, W_fc_1,
           W_g1, b_g1, W_gout, b_gout, W_lin, b_lin):
  n = atom_number.shape[0]
  E, n_dist = distance.shape

  emb_p = _pad2(embedding, 32, 32)
  wcf0_p = _pad2(W_cf_0, 32, 64)
  bcf0_p = _pad1(b_cf_0, 64).reshape(1, 64)
  wdf0_p = _pad2(W_df_0, n_dist, 64)
  wdf0_b = wdf0_p.astype(jnp.bfloat16)
  bdf0_p = _pad1(b_df_0, 64).reshape(1, 64)
  wfc0_p = _pad2(W_fc_0, 64, 32)
  wfc0_b = wfc0_p.astype(jnp.bfloat16)
  wcf1_p = _pad2(W_cf_1, 32, 128)
  bcf1_p = _pad1(b_cf_1, 128).reshape(1, 128)
  wdf1_p = _pad2(W_df_1, n_dist, 64)
  wdf1_b = wdf1_p.astype(jnp.bfloat16)
  bdf1_p = _pad1(b_df_1, 64).reshape(1, 64)
  bdf1_p128 = _pad1(b_df_1, 128).reshape(1, 128)
  wfc1_p = _pad2(W_fc_1, 64, 32)
  wfc1_b = wfc1_p.astype(jnp.bfloat16)
  wfc1_p128 = _pad2(W_fc_1, 128, 32)
  wg1_p = _pad2(W_g1, 32, 128)
  bg1_p = _pad1(b_g1, 128).reshape(1, 128)
  wgout_p = _pad2(W_gout, 128, 16)
  bgout_p = _pad1(b_gout, 16).reshape(1, 16)
  wlint_p = _pad2(W_lin.T, 16, 16)
  blin_p = _pad1(b_lin, 16).reshape(1, 16)

  dmi3 = distance_membership_i.astype(jnp.int32).reshape(E // BLK_E, 1, BLK_E)
  aw_e = (jnp.min(dmi3[:, 0, :], axis=1) // 8) * 8
  nw_e = (jnp.max(dmi3[:, 0, :], axis=1) - aw_e) // WIN_A + 1
  an3 = atom_number.astype(jnp.int32).reshape(n // BLK_N, 1, BLK_N)
  am3 = atom_membership.astype(jnp.int32).reshape(n // BLK_N, 1, BLK_N)
  aw_m = (jnp.min(am3[:, 0, :], axis=1) // 8) * 8
  nw_m = (jnp.max(am3[:, 0, :], axis=1) - aw_m) // WIN_M + 1
  dmj = distance_membership_j.astype(jnp.int32)

  anj = _sc_gather(atom_number.astype(jnp.int32), dmj, 5000)
  anj3 = anj.reshape(E // BLK_E, 1, BLK_E)

  agg0, dist_b = _edge_pass0(distance.T, anj3, dmi3, aw_e, nw_e, emb_p,
                             wcf0_p, bcf0_p, wdf0_b, bdf0_p, wfc0_b, n)
  agg0 = agg0[:n]
  afh1, base1 = _mid_pass(agg0, an3, emb_p, wcf0_p, bcf0_p, bdf0_p, wfc0_p,
                          wcf1_p, bcf1_p, bdf1_p128, wfc1_p128)
  EA = 384000
  gj_a = _sc_gather(afh1, dmj[:EA], 160)
  gj_b = _sc_gather(afh1, dmj[EA:], 200)
  agg1_a = _edge_pass1(dist_b, gj_a, dmi3, aw_e, nw_e, wdf1_b, bdf1_p,
                       wfc1_b, n, 0)
  agg1_b = _edge_pass1(dist_b, gj_b, dmi3, aw_e, nw_e, wdf1_b, bdf1_p,
                       wfc1_b, n, EA // BLK_E)
  agg1 = (agg1_a + agg1_b)[:n]
  out = _final_pass(agg1, base1, am3, aw_m, nw_m, wg1_p, bg1_p, wgout_p,
                    bgout_p, wlint_p, blin_p)
  return out[:N_MOLS, :12]
